# baseline (speedup 1.0000x reference)
import jax
import jax.numpy as jnp
from jax.experimental import pallas as pl


def _body(x_ref, ce_ref, pos_ref, out_ref):
    _, T, rows, C = x_ref.shape
    ce = ce_ref[...]
    pos = pos_ref[...]
    ys = [
        jnp.dot(x_ref[0, t], ce, preferred_element_type=jnp.float32) + pos
        for t in range(T)
    ]
    out_ref[0] = jnp.concatenate(ys, axis=-1)


@jax.jit
def kernel(inputs, channel_embeddings, positional_embeddings):
    B, T, H, W, C = inputs.shape
    _, D = channel_embeddings.shape
    HW = H * W
    hs = 8
    nh = H // hs

    x = inputs.reshape(B, T, HW, C)
    pos = positional_embeddings.reshape(HW, D)

    out = pl.pallas_call(
        _body,
        grid=(B, nh),
        in_specs=[
            pl.BlockSpec((1, T, hs * W, C), lambda b, h: (b, 0, h, 0)),
            pl.BlockSpec((C, D), lambda b, h: (0, 0)),
            pl.BlockSpec((hs * W, D), lambda b, h: (h, 0)),
        ],
        out_specs=pl.BlockSpec((1, hs * W, T * D), lambda b, h: (b, h, 0)),
        out_shape=jax.ShapeDtypeStruct((B, HW, T * D), jnp.float32),
    )(x, channel_embeddings, pos)

    return out.reshape(B, H, W, T, D)


# speedup vs baseline: 1.3324x; 1.3324x over previous
import jax
import jax.numpy as jnp
from jax.experimental import pallas as pl


def _body(x_ref, ce_ref, pos_ref, out_ref):
    _, T, rows, C = x_ref.shape
    _, hs, W, _, D = out_ref.shape
    ce = ce_ref[...]
    pos = pos_ref[...]
    for t in range(T):
        y = jnp.dot(x_ref[0, t], ce, preferred_element_type=jnp.float32)
        out_ref[0, :, :, t, :] = (y + pos).reshape(hs, W, D)


@jax.jit
def kernel(inputs, channel_embeddings, positional_embeddings):
    B, T, H, W, C = inputs.shape
    _, D = channel_embeddings.shape
    HW = H * W
    hs = 8
    nh = H // hs

    x = inputs.reshape(B, T, HW, C)
    pos = positional_embeddings.reshape(HW, D)

    out = pl.pallas_call(
        _body,
        grid=(B, nh),
        in_specs=[
            pl.BlockSpec((1, T, hs * W, C), lambda b, h: (b, 0, h, 0)),
            pl.BlockSpec((C, D), lambda b, h: (0, 0)),
            pl.BlockSpec((hs * W, D), lambda b, h: (h, 0)),
        ],
        out_specs=pl.BlockSpec((1, hs, W, T, D), lambda b, h: (b, h, 0, 0, 0)),
        out_shape=jax.ShapeDtypeStruct((B, H, W, T, D), jnp.float32),
    )(x, channel_embeddings, pos)

    return out


# speedup vs baseline: 2.3567x; 1.7688x over previous
import jax
import jax.numpy as jnp
from jax.experimental import pallas as pl


def _body(x_ref, ce_ref, pos_ref, out_ref):
    _, T, hs, C, W = x_ref.shape
    h = pl.program_id(1)
    ce = ce_ref[...]
    for hi in range(hs):
        p = pos_ref[pl.ds((h * hs + hi) * W, W), :]
        for t in range(T):
            m = x_ref[0, t, hi]
            y = jax.lax.dot_general(
                m, ce, (((0,), (0,)), ((), ())),
                preferred_element_type=jnp.float32)
            out_ref[0, hi, :, t, :] = y + p


@jax.jit
def kernel(inputs, channel_embeddings, positional_embeddings):
    B, T, H, W, C = inputs.shape
    _, D = channel_embeddings.shape
    HW = H * W
    hs = 8
    nh = H // hs

    x = jnp.swapaxes(inputs, 3, 4)
    pos = positional_embeddings.reshape(HW, D)

    out = pl.pallas_call(
        _body,
        grid=(B, nh),
        in_specs=[
            pl.BlockSpec((1, T, hs, C, W), lambda b, h: (b, 0, h, 0, 0)),
            pl.BlockSpec((C, D), lambda b, h: (0, 0)),
            pl.BlockSpec((HW, D), lambda b, h: (0, 0)),
        ],
        out_specs=pl.BlockSpec((1, hs, W, T, D), lambda b, h: (b, h, 0, 0, 0)),
        out_shape=jax.ShapeDtypeStruct((B, H, W, T, D), jnp.float32),
    )(x, channel_embeddings, pos)

    return out


# speedup vs baseline: 2.9496x; 1.2516x over previous
import jax
import jax.numpy as jnp
from jax.experimental import pallas as pl


def _body(x_ref, ce_ref, pos_ref, out_ref):
    _, T, hs, C, W = x_ref.shape
    h = pl.program_id(1)
    ce = ce_ref[...]
    for hi in range(hs):
        p = pos_ref[pl.ds((h * hs + hi) * W, W), :]
        for t in range(T):
            m = x_ref[0, t, hi]
            y = jax.lax.dot_general(
                m, ce, (((0,), (0,)), ((), ())),
                preferred_element_type=jnp.float32)
            out_ref[0, hi, :, t, :] = y + p


@jax.jit
def kernel(inputs, channel_embeddings, positional_embeddings):
    B, T, H, W, C = inputs.shape
    _, D = channel_embeddings.shape
    HW = H * W
    hs = 16
    nh = H // hs

    x = jnp.swapaxes(inputs, 3, 4)
    pos = positional_embeddings.reshape(HW, D)

    out = pl.pallas_call(
        _body,
        grid=(B, nh),
        in_specs=[
            pl.BlockSpec((1, T, hs, C, W), lambda b, h: (b, 0, h, 0, 0)),
            pl.BlockSpec((C, D), lambda b, h: (0, 0)),
            pl.BlockSpec((HW, D), lambda b, h: (0, 0)),
        ],
        out_specs=pl.BlockSpec((1, hs, W, T, D), lambda b, h: (b, h, 0, 0, 0)),
        out_shape=jax.ShapeDtypeStruct((B, H, W, T, D), jnp.float32),
    )(x, channel_embeddings, pos)

    return out
